# baseline (speedup 1.0000x reference)
import functools

import jax
import jax.numpy as jnp
from jax import lax
from jax.experimental import pallas as pl
from jax.experimental.pallas import tpu as pltpu

CAM_DIM = 3
ROT_DIM = 6
N_JOINTS = 22
CENTER2D_THRESH_SQ = 25
POSE_THRESH = 2.5

_NP = 1024
_TI = 8

_PI = 3.14159265358979
_PI_2 = 1.5707963267948966


def _asin_poly(z):
    p = jnp.float32(4.2163199048e-2)
    p = p * z + jnp.float32(2.4181311049e-2)
    p = p * z + jnp.float32(4.5470025998e-2)
    p = p * z + jnp.float32(7.4953002686e-2)
    p = p * z + jnp.float32(1.6666752422e-1)
    return p


def _acos(x):
    ax = jnp.abs(x)
    asin_small = _asin_poly(ax * ax) * (ax * ax) * ax + ax
    acos_mid = jnp.where(x >= 0, _PI_2 - asin_small, _PI_2 + asin_small)
    z = 0.5 * (1.0 - ax)
    s = jnp.sqrt(z)
    asin_s = _asin_poly(z) * z * s + s
    acos_big = jnp.where(x >= 0, 2.0 * asin_s, _PI - 2.0 * asin_s)
    return jnp.where(ax <= 0.5, acos_mid, acos_big)


def _stage1_body(p6_ref, aa_ref):
    for j in range(N_JOINTS):
        b = 6 * j
        a1x = p6_ref[b + 0:b + 1, :]
        a1y = p6_ref[b + 1:b + 2, :]
        a1z = p6_ref[b + 2:b + 3, :]
        a2x = p6_ref[b + 3:b + 4, :]
        a2y = p6_ref[b + 4:b + 5, :]
        a2z = p6_ref[b + 5:b + 6, :]
        n1 = jnp.sqrt(a1x * a1x + a1y * a1y + a1z * a1z)
        inv1 = 1.0 / (n1 + 1e-8)
        b1x = a1x * inv1
        b1y = a1y * inv1
        b1z = a1z * inv1
        d = b1x * a2x + b1y * a2y + b1z * a2z
        ux = a2x - d * b1x
        uy = a2y - d * b1y
        uz = a2z - d * b1z
        n2 = jnp.sqrt(ux * ux + uy * uy + uz * uz)
        inv2 = 1.0 / (n2 + 1e-8)
        b2x = ux * inv2
        b2y = uy * inv2
        b2z = uz * inv2
        b3x = b1y * b2z - b1z * b2y
        b3y = b1z * b2x - b1x * b2z
        b3z = b1x * b2y - b1y * b2x
        tr = b1x + b2y + b3z
        cos = jnp.clip((tr - 1.0) * 0.5, -1.0 + 1e-6, 1.0 - 1e-6)
        ang = _acos(cos)
        sinang = jnp.sqrt((1.0 - cos) * (1.0 + cos))
        f = ang / (2.0 * sinang + 1e-8)
        aa_ref[3 * j + 0:3 * j + 1, :] = (b2z - b3y) * f
        aa_ref[3 * j + 1:3 * j + 2, :] = (b3x - b1z) * f
        aa_ref[3 * j + 2:3 * j + 3, :] = (b1y - b2x) * f


def _stage2_body(aaI_ref, aaT_ref, czI_ref, czT_ref, tsRow_ref, tsI_ref,
                 par_ref, kp_ref, ks_ref, nms_ref, acc_ref):
    y_i = czI_ref[:, 0:1]
    x_i = czI_ref[:, 1:2]
    b_i = czI_ref[:, 2:3]
    dy = czT_ref[0:1, :] - y_i
    dx = czT_ref[1:2, :] - x_i
    d2 = dy * dy + dx * dx
    simc = (d2 <= CENTER2D_THRESH_SQ) & (czT_ref[2:3, :] == b_i)
    simf = jnp.where(simc, 1.0, 0.0).astype(jnp.float32)
    cnt = jnp.sum(simf, axis=1, keepdims=True)
    needs = cnt > 1.0

    @pl.when(jnp.max(cnt) > 1.0)
    def _pose():
        acc = jnp.zeros((_TI, _NP), jnp.float32)
        for c0 in range(0, 3 * N_JOINTS, 3):
            d0 = aaI_ref[:, c0 + 0:c0 + 1] - aaT_ref[c0 + 0:c0 + 1, :]
            d1 = aaI_ref[:, c0 + 1:c0 + 2] - aaT_ref[c0 + 1:c0 + 2, :]
            d2j = aaI_ref[:, c0 + 2:c0 + 3] - aaT_ref[c0 + 2:c0 + 3, :]
            acc = acc + jnp.sqrt(d0 * d0 + d1 * d1 + d2j * d2j + 1e-8)
        acc_ref[...] = acc * (1.0 / N_JOINTS)

    pose_err = acc_ref[...]
    posef = jnp.where(pose_err < POSE_THRESH, 1.0, 0.0).astype(jnp.float32)
    sim = simf * jnp.where(needs, posef, 1.0)
    score = sim * tsRow_ref[...]
    rowmax = jnp.max(score, axis=1, keepdims=True)
    lane = lax.broadcasted_iota(jnp.int32, (_TI, _NP), 1)
    arg = jnp.min(jnp.where(score == rowmax, lane, _NP), axis=1, keepdims=True)
    ig = pl.program_id(0) * _TI + lax.broadcasted_iota(jnp.int32, (_TI, 1), 0)
    keep = arg == ig
    maskf = jnp.where(keep, 1.0, 0.0).astype(jnp.float32)
    kp_ref[...] = par_ref[...] * maskf
    ks_ref[...] = tsI_ref[...] * maskf
    nms_ref[...] = jnp.where(keep, 1, 0).astype(jnp.int32)


@jax.jit
def kernel(params_preds, pred_batch_ids, pred_czyxs, top_score):
    N, D = params_preds.shape
    f32 = jnp.float32
    i32 = jnp.int32

    pose6d = params_preds[:, CAM_DIM:CAM_DIM + N_JOINTS * ROT_DIM]
    p6T = jnp.zeros((N_JOINTS * ROT_DIM, _NP), f32).at[:, :N].set(pose6d.T)

    aaT = pl.pallas_call(
        _stage1_body,
        out_shape=jax.ShapeDtypeStruct((3 * N_JOINTS, _NP), f32),
    )(p6T)
    aaI = aaT.T

    czy = pred_czyxs[:, 1].astype(i32)
    czx = pred_czyxs[:, 2].astype(i32)
    bid = pred_batch_ids.astype(i32)
    czI = jnp.full((_NP, 4), -1, i32)
    czI = czI.at[:N, 0].set(czy).at[:N, 1].set(czx).at[:N, 2].set(bid)
    czT = jnp.full((8, _NP), -1, i32)
    czT = czT.at[0, :N].set(czy).at[1, :N].set(czx).at[2, :N].set(bid)
    tsRow = jnp.zeros((1, _NP), f32).at[0, :N].set(top_score)
    tsI = jnp.zeros((_NP, 1), f32).at[:N, 0].set(top_score)
    par = jnp.zeros((_NP, D), f32).at[:N, :].set(params_preds)

    grid = _NP // _TI
    kp, ks, nms = pl.pallas_call(
        _stage2_body,
        grid=(grid,),
        in_specs=[
            pl.BlockSpec((_TI, 3 * N_JOINTS), lambda t: (t, 0)),
            pl.BlockSpec((3 * N_JOINTS, _NP), lambda t: (0, 0)),
            pl.BlockSpec((_TI, 4), lambda t: (t, 0)),
            pl.BlockSpec((8, _NP), lambda t: (0, 0)),
            pl.BlockSpec((1, _NP), lambda t: (0, 0)),
            pl.BlockSpec((_TI, 1), lambda t: (t, 0)),
            pl.BlockSpec((_TI, D), lambda t: (t, 0)),
        ],
        out_specs=[
            pl.BlockSpec((_TI, D), lambda t: (t, 0)),
            pl.BlockSpec((_TI, 1), lambda t: (t, 0)),
            pl.BlockSpec((_TI, 1), lambda t: (t, 0)),
        ],
        out_shape=[
            jax.ShapeDtypeStruct((_NP, D), f32),
            jax.ShapeDtypeStruct((_NP, 1), f32),
            jax.ShapeDtypeStruct((_NP, 1), i32),
        ],
        scratch_shapes=[pltpu.VMEM((_TI, _NP), f32)],
    )(aaI, aaT, czI, czT, tsRow, tsI, par)

    return kp[:N, :], ks[:N, 0], nms[:N, 0].astype(jnp.bool_)
